# baseline (speedup 1.0000x reference)
import jax
import jax.numpy as jnp
from jax import lax
from jax.experimental import pallas as pl
from jax.experimental.pallas import tpu as pltpu
from jax.experimental.pallas import tpu_sc as plsc

VOCAB = 1000000
EMB = 64
LANES = 16
NC, NS = 2, 16
NW = NC * NS
BATCH = 16384
HIST = 50
BW = BATCH // NW
K1_W = 16384
K1_GRID = (VOCAB + K1_W - 1) // K1_W
T2_ROWS = K1_GRID * (K1_W // 2)


def _repack_body(w_ref, out_ref):
    x = w_ref[...]
    out_ref[:, :EMB] = x[:, : K1_W // 2].T * 8.0
    out_ref[:, EMB:] = x[:, K1_W // 2:].T * 8.0


def _make_pairs_table(w_t):
    return pl.pallas_call(
        _repack_body,
        grid=(K1_GRID,),
        in_specs=[pl.BlockSpec((EMB, K1_W), lambda i: (0, i))],
        out_specs=pl.BlockSpec((K1_W // 2, 2 * EMB), lambda i: (i, 0)),
        out_shape=jax.ShapeDtypeStruct((T2_ROWS, 2 * EMB), jnp.float32),
    )(w_t)


CW = 128
GP = 131
NCH = HIST * (BW // CW)
NG = CW // LANES


def _lookup_body(idx_hbm, tab_hbm, out_hbm,
                 idx_all, idx_t0, idx_t1, pc0, pc1, g0, g1, ov0, ov1, gsem, wsem):
    wid = lax.axis_index("s") * NC + lax.axis_index("c")
    b0 = wid * BW
    lanes = lax.iota(jnp.int32, LANES)
    idx_t = (idx_t0, idx_t1)
    pcol = (pc0, pc1)
    g_v = (g0, g1)
    out_v = (ov0, ov1)

    pltpu.sync_copy(idx_hbm.at[:, pl.ds(b0, BW)], idx_all)

    def prep(t, b):
        h = t // (BW // CW)
        c0 = (t % (BW // CW)) * CW

        @plsc.parallel_loop(0, CW // LANES, unroll=4)
        def pstep(j):
            v = idx_all[h, pl.ds(c0 + j * LANES, LANES)]
            idx_t[b][pl.ds(j * LANES, LANES)] = (
                ((v >> 14) << 13) | (v & (K1_W // 2 - 1)))
            pcol[b][pl.ds(j * LANES, LANES)] = (v >> 7) & EMB

    def consume(t, b):
        h = t // (BW // CW)
        c0 = (t % (BW // CW)) * CW
        rows = [lanes + g * LANES for g in range(NG)]
        pcs = [pcol[b][pl.ds(g * LANES, LANES)] for g in range(NG)]

        @plsc.parallel_loop(0, EMB, unroll=6)
        def e_step(e):
            for g in range(NG):
                val = plsc.load_gather(g_v[b], [rows[g], pcs[g] + e])
                out_v[b][e, pl.ds(g * LANES, LANES)] = val
        pltpu.async_copy(out_v[b], out_hbm.at[h, :, pl.ds(b0 + c0, CW)], wsem)

    prep(0, 0)
    pltpu.async_copy(tab_hbm.at[idx_t[0]], g_v[0].at[:, pl.ds(0, 2 * EMB)], gsem)

    def pair_step(p, carry):
        for b in (0, 1):
            t = 2 * p + b
            nb = 1 - b

            @pl.when(t + 1 < NCH)
            def _():
                prep(t + 1, nb)
                pltpu.async_copy(tab_hbm.at[idx_t[nb]], g_v[nb].at[:, pl.ds(0, 2 * EMB)], gsem)

            pltpu.make_async_copy(tab_hbm.at[idx_t[b]], g_v[b].at[:, pl.ds(0, 2 * EMB)], gsem).wait()

            @pl.when(t >= 2)
            def _():
                h2 = (t - 2) // (BW // CW)
                c2 = ((t - 2) % (BW // CW)) * CW
                pltpu.make_async_copy(
                    out_v[b], out_hbm.at[h2, :, pl.ds(b0 + c2, CW)], wsem).wait()

            consume(t, b)
        return carry

    lax.fori_loop(0, NCH // 2, pair_step, 0)

    for t in (NCH - 2, NCH - 1):
        hl = t // (BW // CW)
        cl = (t % (BW // CW)) * CW
        pltpu.make_async_copy(
            out_v[t % 2], out_hbm.at[hl, :, pl.ds(b0 + cl, CW)], wsem).wait()


def kernel(input, weight):
    w_t = weight.T
    tab = _make_pairs_table(w_t)
    in_t = input.T.astype(jnp.int32)

    mesh = plsc.VectorSubcoreMesh(core_axis_name="c", subcore_axis_name="s")
    run = pl.kernel(
        _lookup_body,
        out_type=jax.ShapeDtypeStruct((HIST, EMB, BATCH), jnp.float32),
        mesh=mesh,
        scratch_types=[
            pltpu.VMEM((HIST, BW), jnp.int32),
            pltpu.VMEM((CW,), jnp.int32),
            pltpu.VMEM((CW,), jnp.int32),
            pltpu.VMEM((CW,), jnp.int32),
            pltpu.VMEM((CW,), jnp.int32),
            pltpu.VMEM((CW, GP), jnp.float32),
            pltpu.VMEM((CW, GP), jnp.float32),
            pltpu.VMEM((EMB, CW), jnp.float32),
            pltpu.VMEM((EMB, CW), jnp.float32),
            pltpu.SemaphoreType.DMA,
            pltpu.SemaphoreType.DMA,
        ],
        compiler_params=pltpu.CompilerParams(
            use_tc_tiling_on_sc=True, needs_layout_passes=False),
    )
    a = run(in_t, tab)
    return a.transpose(2, 0, 1)


# speedup vs baseline: 1.1258x; 1.1258x over previous
import jax
import jax.numpy as jnp
from jax import lax
from jax.experimental import pallas as pl
from jax.experimental.pallas import tpu as pltpu
from jax.experimental.pallas import tpu_sc as plsc

VOCAB = 1000000
EMB = 64
LANES = 16
NC, NS = 2, 16
NW = NC * NS
BATCH = 16384
HIST = 50
BW = BATCH // NW
K1_W = 16384
K1_GRID = (VOCAB + K1_W - 1) // K1_W
T2_ROWS = K1_GRID * (K1_W // 2)


def _repack_body(w_ref, out_ref):
    x = w_ref[...]
    out_ref[:, :EMB] = x[:, : K1_W // 2].T * 8.0
    out_ref[:, EMB:] = x[:, K1_W // 2:].T * 8.0


def _make_pairs_table(w_t):
    return pl.pallas_call(
        _repack_body,
        grid=(K1_GRID,),
        in_specs=[pl.BlockSpec((EMB, K1_W), lambda i: (0, i))],
        out_specs=pl.BlockSpec((K1_W // 2, 2 * EMB), lambda i: (i, 0)),
        out_shape=jax.ShapeDtypeStruct((T2_ROWS, 2 * EMB), jnp.float32),
    )(w_t)


CW = 128
GP = 131
NCH = HIST * (BW // CW)
NG = CW // LANES


def _lookup_body(idx_hbm, tab_hbm, out_hbm,
                 idx_all, idx_t0, idx_t1, pc0, pc1, g0, g1, ov0, ov1, gsem, wsem):
    wid = lax.axis_index("s") * NC + lax.axis_index("c")
    b0 = wid * BW
    lanes = lax.iota(jnp.int32, LANES)
    idx_t = (idx_t0, idx_t1)
    pcol = (pc0, pc1)
    g_v = (g0, g1)
    out_v = (ov0, ov1)

    pltpu.sync_copy(idx_hbm.at[:, pl.ds(b0, BW)], idx_all)

    def prep(t, b):
        h = t // (BW // CW)
        c0 = (t % (BW // CW)) * CW

        @plsc.parallel_loop(0, CW // LANES, unroll=4)
        def pstep(j):
            v = idx_all[h, pl.ds(c0 + j * LANES, LANES)]
            idx_t[b][pl.ds(j * LANES, LANES)] = (
                ((v >> 14) << 13) | (v & (K1_W // 2 - 1)))
            pcol[b][pl.ds(j * LANES, LANES)] = (v >> 7) & EMB

    def consume(t, b):
        h = t // (BW // CW)
        c0 = (t % (BW // CW)) * CW
        rows = [lanes + g * LANES for g in range(NG)]
        pcs = [pcol[b][pl.ds(g * LANES, LANES)] for g in range(NG)]

        @plsc.parallel_loop(0, EMB, unroll=4)
        def e_step(e):
            for g in range(NG):
                val = plsc.load_gather(g_v[b], [rows[g], pcs[g] + e])
                out_v[b][e, pl.ds(g * LANES, LANES)] = val
        pltpu.async_copy(out_v[b], out_hbm.at[h, :, pl.ds(b0 + c0, CW)], wsem)

    prep(0, 0)
    pltpu.async_copy(tab_hbm.at[idx_t[0]], g_v[0].at[:, pl.ds(0, 2 * EMB)], gsem)

    def pair_step(p, carry):
        for b in (0, 1):
            t = 2 * p + b
            nb = 1 - b

            @pl.when(t + 1 < NCH)
            def _():
                prep(t + 1, nb)
                pltpu.async_copy(tab_hbm.at[idx_t[nb]], g_v[nb].at[:, pl.ds(0, 2 * EMB)], gsem)

            pltpu.make_async_copy(tab_hbm.at[idx_t[b]], g_v[b].at[:, pl.ds(0, 2 * EMB)], gsem).wait()

            @pl.when(t >= 2)
            def _():
                h2 = (t - 2) // (BW // CW)
                c2 = ((t - 2) % (BW // CW)) * CW
                pltpu.make_async_copy(
                    out_v[b], out_hbm.at[h2, :, pl.ds(b0 + c2, CW)], wsem).wait()

            consume(t, b)
        return carry

    lax.fori_loop(0, NCH // 2, pair_step, 0)

    for t in (NCH - 2, NCH - 1):
        hl = t // (BW // CW)
        cl = (t % (BW // CW)) * CW
        pltpu.make_async_copy(
            out_v[t % 2], out_hbm.at[hl, :, pl.ds(b0 + cl, CW)], wsem).wait()


def kernel(input, weight):
    w_t = weight.T
    tab = _make_pairs_table(w_t)
    in_t = input.T.astype(jnp.int32)

    mesh = plsc.VectorSubcoreMesh(core_axis_name="c", subcore_axis_name="s")
    run = pl.kernel(
        _lookup_body,
        out_type=jax.ShapeDtypeStruct((HIST, EMB, BATCH), jnp.float32),
        mesh=mesh,
        scratch_types=[
            pltpu.VMEM((HIST, BW), jnp.int32),
            pltpu.VMEM((CW,), jnp.int32),
            pltpu.VMEM((CW,), jnp.int32),
            pltpu.VMEM((CW,), jnp.int32),
            pltpu.VMEM((CW,), jnp.int32),
            pltpu.VMEM((CW, GP), jnp.float32),
            pltpu.VMEM((CW, GP), jnp.float32),
            pltpu.VMEM((EMB, CW), jnp.float32),
            pltpu.VMEM((EMB, CW), jnp.float32),
            pltpu.SemaphoreType.DMA,
            pltpu.SemaphoreType.DMA,
        ],
        compiler_params=pltpu.CompilerParams(
            use_tc_tiling_on_sc=True, needs_layout_passes=False),
    )
    a = run(in_t, tab)
    return a.transpose(2, 0, 1)


# speedup vs baseline: 2.0148x; 1.7896x over previous
import jax
import jax.numpy as jnp
from jax import lax
from jax.experimental import pallas as pl
from jax.experimental.pallas import tpu as pltpu
from jax.experimental.pallas import tpu_sc as plsc

VOCAB = 1000000
EMB = 64
LANES = 16
NC, NS = 2, 16
NW = NC * NS
BATCH = 16384
HIST = 50
BW = BATCH // NW
K1_W = 16384
K1_GRID = (VOCAB + K1_W - 1) // K1_W
T2_ROWS = K1_GRID * (K1_W // 2)


def _repack_body(w_ref, out_ref):
    x = w_ref[...]
    out_ref[:, :EMB] = x[:, : K1_W // 2].T * 8.0
    out_ref[:, EMB:] = x[:, K1_W // 2:].T * 8.0


def _make_pairs_table(w_t):
    return pl.pallas_call(
        _repack_body,
        grid=(K1_GRID,),
        in_specs=[pl.BlockSpec((EMB, K1_W), lambda i: (0, i))],
        out_specs=pl.BlockSpec((K1_W // 2, 2 * EMB), lambda i: (i, 0)),
        out_shape=jax.ShapeDtypeStruct((T2_ROWS, 2 * EMB), jnp.float32),
    )(w_t)


CW = 128
GP = 131
NCH = HIST * (BW // CW)
NG = CW // LANES


def _lookup_body(idx_hbm, tab_hbm, out_hbm,
                 idx_all, idx_t0, idx_t1, pc0, pc1, g0, g1, ov0, ov1, gsem, wsem):
    wid = lax.axis_index("s") * NC + lax.axis_index("c")
    b0 = wid * BW
    lanes = lax.iota(jnp.int32, LANES)
    idx_t = (idx_t0, idx_t1)
    pcol = (pc0, pc1)
    g_v = (g0, g1)
    out_v = (ov0, ov1)

    pltpu.sync_copy(idx_hbm.at[:, pl.ds(b0, BW)], idx_all)

    def prep(t, b):
        h = t // (BW // CW)
        c0 = (t % (BW // CW)) * CW

        @plsc.parallel_loop(0, CW // LANES, unroll=4)
        def pstep(j):
            v = idx_all[h, pl.ds(c0 + j * LANES, LANES)]
            idx_t[b][pl.ds(j * LANES, LANES)] = (
                ((v >> 14) << 13) | (v & (K1_W // 2 - 1)))
            pcol[b][pl.ds(j * LANES, LANES)] = (v >> 7) & EMB

    def consume(t, b):
        h = t // (BW // CW)
        c0 = (t % (BW // CW)) * CW
        for g in range(NG):
            bvec = lanes + g * LANES
            pc = pcol[b][pl.ds(g * LANES, LANES)]

            @plsc.parallel_loop(0, EMB, unroll=4)
            def ed_step(i):
                rvec = (i & ~(LANES - 1)) + ((lanes + i) & (LANES - 1))
                val = plsc.load_gather(g_v[b], [bvec, pc + rvec])
                plsc.store_scatter(out_v[b], [rvec, bvec], val)
        pltpu.async_copy(out_v[b], out_hbm.at[h, :, pl.ds(b0 + c0, CW)], wsem)

    prep(0, 0)
    pltpu.async_copy(tab_hbm.at[idx_t[0]], g_v[0].at[:, pl.ds(0, 2 * EMB)], gsem)

    def pair_step(p, carry):
        for b in (0, 1):
            t = 2 * p + b
            nb = 1 - b

            @pl.when(t + 1 < NCH)
            def _():
                prep(t + 1, nb)
                pltpu.async_copy(tab_hbm.at[idx_t[nb]], g_v[nb].at[:, pl.ds(0, 2 * EMB)], gsem)

            pltpu.make_async_copy(tab_hbm.at[idx_t[b]], g_v[b].at[:, pl.ds(0, 2 * EMB)], gsem).wait()

            @pl.when(t >= 2)
            def _():
                h2 = (t - 2) // (BW // CW)
                c2 = ((t - 2) % (BW // CW)) * CW
                pltpu.make_async_copy(
                    out_v[b], out_hbm.at[h2, :, pl.ds(b0 + c2, CW)], wsem).wait()

            consume(t, b)
        return carry

    lax.fori_loop(0, NCH // 2, pair_step, 0)

    for t in (NCH - 2, NCH - 1):
        hl = t // (BW // CW)
        cl = (t % (BW // CW)) * CW
        pltpu.make_async_copy(
            out_v[t % 2], out_hbm.at[hl, :, pl.ds(b0 + cl, CW)], wsem).wait()


def kernel(input, weight):
    w_t = weight.T
    tab = _make_pairs_table(w_t)
    in_t = input.T.astype(jnp.int32)

    mesh = plsc.VectorSubcoreMesh(core_axis_name="c", subcore_axis_name="s")
    run = pl.kernel(
        _lookup_body,
        out_type=jax.ShapeDtypeStruct((HIST, EMB, BATCH), jnp.float32),
        mesh=mesh,
        scratch_types=[
            pltpu.VMEM((HIST, BW), jnp.int32),
            pltpu.VMEM((CW,), jnp.int32),
            pltpu.VMEM((CW,), jnp.int32),
            pltpu.VMEM((CW,), jnp.int32),
            pltpu.VMEM((CW,), jnp.int32),
            pltpu.VMEM((CW, GP), jnp.float32),
            pltpu.VMEM((CW, GP), jnp.float32),
            pltpu.VMEM((EMB, CW), jnp.float32),
            pltpu.VMEM((EMB, CW), jnp.float32),
            pltpu.SemaphoreType.DMA,
            pltpu.SemaphoreType.DMA,
        ],
        compiler_params=pltpu.CompilerParams(
            use_tc_tiling_on_sc=True, needs_layout_passes=False),
    )
    a = run(in_t, tab)
    return a.transpose(2, 0, 1)


# speedup vs baseline: 2.0706x; 1.0277x over previous
import jax
import jax.numpy as jnp
from jax import lax
from jax.experimental import pallas as pl
from jax.experimental.pallas import tpu as pltpu
from jax.experimental.pallas import tpu_sc as plsc

VOCAB = 1000000
EMB = 64
LANES = 16
NC, NS = 2, 16
NW = NC * NS
BATCH = 16384
HIST = 50
BW = BATCH // NW
K1_W = 32768
K1_GRID = (VOCAB + K1_W - 1) // K1_W
T2_ROWS = K1_GRID * (K1_W // 2)


def _repack_body(w_ref, out_ref):
    x = w_ref[...]
    out_ref[:, :EMB] = x[:, : K1_W // 2].T * 8.0
    out_ref[:, EMB:] = x[:, K1_W // 2:].T * 8.0


def _make_pairs_table(w_t):
    return pl.pallas_call(
        _repack_body,
        grid=(K1_GRID,),
        in_specs=[pl.BlockSpec((EMB, K1_W), lambda i: (0, i))],
        out_specs=pl.BlockSpec((K1_W // 2, 2 * EMB), lambda i: (i, 0)),
        out_shape=jax.ShapeDtypeStruct((T2_ROWS, 2 * EMB), jnp.float32),
    )(w_t)


CW = 128
GP = 131
NCH = HIST * (BW // CW)
NG = CW // LANES


def _lookup_body(idx_hbm, tab_hbm, out_hbm,
                 idx_all, idx_t0, idx_t1, pc0, pc1, g0, g1, ov0, ov1, gsem, wsem):
    wid = lax.axis_index("s") * NC + lax.axis_index("c")
    b0 = wid * BW
    lanes = lax.iota(jnp.int32, LANES)
    idx_t = (idx_t0, idx_t1)
    pcol = (pc0, pc1)
    g_v = (g0, g1)
    out_v = (ov0, ov1)

    pltpu.sync_copy(idx_hbm.at[:, pl.ds(b0, BW)], idx_all)

    def prep(t, b):
        h = t // (BW // CW)
        c0 = (t % (BW // CW)) * CW

        @plsc.parallel_loop(0, CW // LANES, unroll=4)
        def pstep(j):
            v = idx_all[h, pl.ds(c0 + j * LANES, LANES)]
            idx_t[b][pl.ds(j * LANES, LANES)] = (
                ((v >> 15) << 14) | (v & (K1_W // 2 - 1)))
            pcol[b][pl.ds(j * LANES, LANES)] = (v >> 8) & EMB

    def consume(t, b):
        h = t // (BW // CW)
        c0 = (t % (BW // CW)) * CW
        for g in range(NG):
            bvec = lanes + g * LANES
            pc = pcol[b][pl.ds(g * LANES, LANES)]

            @plsc.parallel_loop(0, EMB, unroll=4)
            def ed_step(i):
                rvec = (i & ~(LANES - 1)) + ((lanes + i) & (LANES - 1))
                val = plsc.load_gather(g_v[b], [bvec, pc + rvec])
                plsc.store_scatter(out_v[b], [rvec, bvec], val)
        pltpu.async_copy(out_v[b], out_hbm.at[h, :, pl.ds(b0 + c0, CW)], wsem)

    prep(0, 0)
    pltpu.async_copy(tab_hbm.at[idx_t[0]], g_v[0].at[:, pl.ds(0, 2 * EMB)], gsem)

    def pair_step(p, carry):
        for b in (0, 1):
            t = 2 * p + b
            nb = 1 - b

            @pl.when(t + 1 < NCH)
            def _():
                prep(t + 1, nb)
                pltpu.async_copy(tab_hbm.at[idx_t[nb]], g_v[nb].at[:, pl.ds(0, 2 * EMB)], gsem)

            pltpu.make_async_copy(tab_hbm.at[idx_t[b]], g_v[b].at[:, pl.ds(0, 2 * EMB)], gsem).wait()

            @pl.when(t >= 2)
            def _():
                h2 = (t - 2) // (BW // CW)
                c2 = ((t - 2) % (BW // CW)) * CW
                pltpu.make_async_copy(
                    out_v[b], out_hbm.at[h2, :, pl.ds(b0 + c2, CW)], wsem).wait()

            consume(t, b)
        return carry

    lax.fori_loop(0, NCH // 2, pair_step, 0)

    for t in (NCH - 2, NCH - 1):
        hl = t // (BW // CW)
        cl = (t % (BW // CW)) * CW
        pltpu.make_async_copy(
            out_v[t % 2], out_hbm.at[hl, :, pl.ds(b0 + cl, CW)], wsem).wait()


def kernel(input, weight):
    w_t = weight.T
    tab = _make_pairs_table(w_t)
    in_t = input.T.astype(jnp.int32)

    mesh = plsc.VectorSubcoreMesh(core_axis_name="c", subcore_axis_name="s")
    run = pl.kernel(
        _lookup_body,
        out_type=jax.ShapeDtypeStruct((HIST, EMB, BATCH), jnp.float32),
        mesh=mesh,
        scratch_types=[
            pltpu.VMEM((HIST, BW), jnp.int32),
            pltpu.VMEM((CW,), jnp.int32),
            pltpu.VMEM((CW,), jnp.int32),
            pltpu.VMEM((CW,), jnp.int32),
            pltpu.VMEM((CW,), jnp.int32),
            pltpu.VMEM((CW, GP), jnp.float32),
            pltpu.VMEM((CW, GP), jnp.float32),
            pltpu.VMEM((EMB, CW), jnp.float32),
            pltpu.VMEM((EMB, CW), jnp.float32),
            pltpu.SemaphoreType.DMA,
            pltpu.SemaphoreType.DMA,
        ],
        compiler_params=pltpu.CompilerParams(
            use_tc_tiling_on_sc=True, needs_layout_passes=False),
    )
    a = run(in_t, tab)
    return a.transpose(2, 0, 1)


# speedup vs baseline: 2.1515x; 1.0391x over previous
import jax
import jax.numpy as jnp
from jax import lax
from jax.experimental import pallas as pl
from jax.experimental.pallas import tpu as pltpu
from jax.experimental.pallas import tpu_sc as plsc

VOCAB = 1000000
EMB = 64
LANES = 16
NC, NS = 2, 16
NW = NC * NS
BATCH = 16384
HIST = 50
BW = BATCH // NW
K1_W = 32768
K1_GRID = (VOCAB + K1_W - 1) // K1_W
T2_ROWS = K1_GRID * (K1_W // 2)


def _repack_body(w_ref, out_ref):
    x = w_ref[...]
    out_ref[:, :EMB] = x[:, : K1_W // 2].T * 8.0
    out_ref[:, EMB:] = x[:, K1_W // 2:].T * 8.0


def _make_pairs_table(w_t):
    return pl.pallas_call(
        _repack_body,
        grid=(K1_GRID,),
        in_specs=[pl.BlockSpec((EMB, K1_W), lambda i: (0, i))],
        out_specs=pl.BlockSpec((K1_W // 2, 2 * EMB), lambda i: (i, 0)),
        out_shape=jax.ShapeDtypeStruct((T2_ROWS, 2 * EMB), jnp.float32),
    )(w_t)


CW = 256
GP = 128
NCH = HIST * (BW // CW)
NG = CW // LANES


def _lookup_body(idx_hbm, tab_hbm, out_hbm,
                 idx_all, idx_t0, idx_t1, pc0, pc1, g0, g1, ov0, ov1, gsem, wsem):
    wid = lax.axis_index("s") * NC + lax.axis_index("c")
    b0 = wid * BW
    lanes = lax.iota(jnp.int32, LANES)
    idx_t = (idx_t0, idx_t1)
    pcol = (pc0, pc1)
    g_v = (g0, g1)
    out_v = (ov0, ov1)

    pltpu.sync_copy(idx_hbm.at[:, pl.ds(b0, BW)], idx_all)

    def prep(t, b):
        h = t // (BW // CW)
        c0 = (t % (BW // CW)) * CW

        @plsc.parallel_loop(0, CW // LANES, unroll=4)
        def pstep(j):
            v = idx_all[h, pl.ds(c0 + j * LANES, LANES)]
            idx_t[b][pl.ds(j * LANES, LANES)] = (
                ((v >> 15) << 14) | (v & (K1_W // 2 - 1)))
            pcol[b][pl.ds(j * LANES, LANES)] = (v >> 8) & EMB

    def consume(t, b):
        h = t // (BW // CW)
        c0 = (t % (BW // CW)) * CW
        for g in range(NG):
            bvec = lanes + g * LANES
            pc = pcol[b][pl.ds(g * LANES, LANES)]

            @plsc.parallel_loop(0, EMB, unroll=4)
            def ed_step(i):
                rvec = (i & ~(LANES - 1)) + ((lanes + i) & (LANES - 1))
                val = plsc.load_gather(g_v[b], [bvec, pc + rvec])
                plsc.store_scatter(out_v[b], [rvec, bvec], val)
        pltpu.async_copy(out_v[b], out_hbm.at[h, :, pl.ds(b0 + c0, CW)], wsem)

    prep(0, 0)
    pltpu.async_copy(tab_hbm.at[idx_t[0]], g_v[0], gsem)

    def pair_step(p, carry):
        for b in (0, 1):
            t = 2 * p + b
            nb = 1 - b

            @pl.when(t + 1 < NCH)
            def _():
                prep(t + 1, nb)
                pltpu.async_copy(tab_hbm.at[idx_t[nb]], g_v[nb], gsem)

            pltpu.make_async_copy(tab_hbm.at[idx_t[b]], g_v[b], gsem).wait()

            @pl.when(t >= 2)
            def _():
                h2 = (t - 2) // (BW // CW)
                c2 = ((t - 2) % (BW // CW)) * CW
                pltpu.make_async_copy(
                    out_v[b], out_hbm.at[h2, :, pl.ds(b0 + c2, CW)], wsem).wait()

            consume(t, b)
        return carry

    lax.fori_loop(0, NCH // 2, pair_step, 0)

    for t in (NCH - 2, NCH - 1):
        hl = t // (BW // CW)
        cl = (t % (BW // CW)) * CW
        pltpu.make_async_copy(
            out_v[t % 2], out_hbm.at[hl, :, pl.ds(b0 + cl, CW)], wsem).wait()


def kernel(input, weight):
    w_t = weight.T
    tab = _make_pairs_table(w_t)
    in_t = input.T.astype(jnp.int32)

    mesh = plsc.VectorSubcoreMesh(core_axis_name="c", subcore_axis_name="s")
    run = pl.kernel(
        _lookup_body,
        out_type=jax.ShapeDtypeStruct((HIST, EMB, BATCH), jnp.float32),
        mesh=mesh,
        scratch_types=[
            pltpu.VMEM((HIST, BW), jnp.int32),
            pltpu.VMEM((CW,), jnp.int32),
            pltpu.VMEM((CW,), jnp.int32),
            pltpu.VMEM((CW,), jnp.int32),
            pltpu.VMEM((CW,), jnp.int32),
            pltpu.VMEM((CW, GP), jnp.float32),
            pltpu.VMEM((CW, GP), jnp.float32),
            pltpu.VMEM((EMB, CW), jnp.float32),
            pltpu.VMEM((EMB, CW), jnp.float32),
            pltpu.SemaphoreType.DMA,
            pltpu.SemaphoreType.DMA,
        ],
        compiler_params=pltpu.CompilerParams(
            use_tc_tiling_on_sc=True, needs_layout_passes=False),
    )
    a = run(in_t, tab)
    return a.transpose(2, 0, 1)
